# baseline (speedup 1.0000x reference)
import functools

import jax
import jax.numpy as jnp
from jax import lax
from jax.experimental import pallas as pl
from jax.experimental.pallas import tpu as pltpu
from jax.experimental.pallas import tpu_sc as plsc

N = 10000
NP = 10112
NSUB = 16
RPT = NP // NSUB
EW = 128
R = 400
GRID = N // R


def _mesh():
    return plsc.VectorSubcoreMesh(core_axis_name="c", subcore_axis_name="s")


def _pad_edges(src, dst):
    e_tot = src.shape[0]
    nw = -(-e_tot // (NSUB * EW))
    nw += nw % 2
    ep = NSUB * EW * nw
    pad = ep - e_tot
    pad_src = (jnp.arange(pad, dtype=jnp.int32) * 7919) % N
    pad_dst = N + (jnp.arange(pad, dtype=jnp.int32) % 16)
    src3 = jnp.concatenate([src, pad_src]).reshape(NSUB, nw, EW)
    dst3 = jnp.concatenate([dst, pad_dst]).reshape(NSUB, nw, EW)
    return src3, dst3, nw


def _sc_degree(dst3, ones128, zeros128):
    nw = dst3.shape[1]
    nwh = nw // 2
    dst4 = dst3.reshape(NSUB, 2, nwh, EW)

    @functools.partial(
        pl.kernel,
        mesh=_mesh(),
        out_type=[jax.ShapeDtypeStruct((NP, 128), jnp.float32)] * 2,
        scratch_types=[
            pltpu.VMEM((nwh, EW), jnp.int32),
            pltpu.VMEM((EW, 128), jnp.float32),
            pltpu.VMEM_SHARED((NP, 128), jnp.float32),
        ],
    )
    def k(dst_hbm, ones_hbm, zeros_hbm, out0_hbm, out1_hbm, dst_v, ones_v, acc):
        core = lax.axis_index("c")
        sub = lax.axis_index("s")
        row0 = sub * RPT

        def half(half_idx, out_hbm):
            pltpu.sync_copy(dst_hbm.at[sub, half_idx], dst_v)
            pltpu.sync_copy(ones_hbm, ones_v)
            pltpu.sync_copy(zeros_hbm.at[pl.ds(row0, RPT)],
                            acc.at[pl.ds(row0, RPT)])
            plsc.subcore_barrier()

            def body(w, carry):
                pltpu.sync_copy(ones_v, acc.at[dst_v.at[w]], add=True)
                return carry

            lax.fori_loop(0, nwh, body, 0)
            plsc.subcore_barrier()
            pltpu.sync_copy(acc.at[pl.ds(row0, RPT)],
                            out_hbm.at[pl.ds(row0, RPT)])

        @pl.when(core == 0)
        def _():
            half(0, out0_hbm)

        @pl.when(core == 1)
        def _():
            half(1, out1_hbm)

    return k(dst4, ones128, zeros128)


def _sc_aggregate(src3, dst3, u_chunks, zeros_hbm):
    nch = len(u_chunks)
    cpc = nch // 2
    nw = src3.shape[1]
    ngrp = 2
    wpg = nw // ngrp
    src4 = src3.reshape(NSUB, ngrp, wpg, EW)
    dst4 = dst3.reshape(NSUB, ngrp, wpg, EW)

    @functools.partial(
        pl.kernel,
        mesh=_mesh(),
        out_type=[jax.ShapeDtypeStruct((NP, 128), jnp.float32)] * nch,
        scratch_types=[
            pltpu.VMEM((wpg, EW), jnp.int32),
            pltpu.VMEM((wpg, EW), jnp.int32),
            pltpu.VMEM((2, EW, 128), jnp.float32),
            pltpu.VMEM_SHARED((NP, 128), jnp.float32),
            pltpu.SemaphoreType.DMA,
            pltpu.SemaphoreType.DMA,
        ],
    )
    def k(*refs):
        src_hbm, dst_hbm = refs[0], refs[1]
        u_refs = refs[2:2 + nch]
        z_hbm = refs[2 + nch]
        out_refs = refs[3 + nch:3 + 2 * nch]
        src_v, dst_v, rows, acc, sem0, sem1 = refs[3 + 2 * nch:]
        sems = (sem0, sem1)
        core = lax.axis_index("c")
        sub = lax.axis_index("s")
        row0 = sub * RPT

        def one_chunk(u_hbm, out_hbm):
            pltpu.sync_copy(z_hbm.at[pl.ds(row0, RPT)],
                            acc.at[pl.ds(row0, RPT)])
            plsc.subcore_barrier()

            def g_start(w, b):
                pltpu.async_copy(u_hbm.at[src_v.at[w]], rows.at[b], sems[b])

            def g_wait(b):
                pltpu.make_async_copy(u_hbm.at[src_v.at[0]], rows.at[b],
                                      sems[b]).wait()

            for g in range(ngrp):
                pltpu.sync_copy(src_hbm.at[sub, g], src_v)
                pltpu.sync_copy(dst_hbm.at[sub, g], dst_v)
                g_start(0, 0)
                g_start(1, 1)

                def body(i, carry):
                    for b in range(2):
                        w = 2 * i + b
                        g_wait(b)
                        pltpu.sync_copy(rows.at[b], acc.at[dst_v.at[w]],
                                        add=True)

                        @pl.when(w + 2 < wpg)
                        def _():
                            g_start(w + 2, b)
                    return carry

                lax.fori_loop(0, wpg // 2, body, 0)
            plsc.subcore_barrier()
            pltpu.sync_copy(acc.at[pl.ds(row0, RPT)],
                            out_hbm.at[pl.ds(row0, RPT)])

        for p in range(cpc):
            @pl.when(core == 0)
            def _(p=p):
                one_chunk(u_refs[p], out_refs[p])

            @pl.when(core == 1)
            def _(p=p):
                one_chunk(u_refs[cpc + p], out_refs[cpc + p])

    return list(k(src4, dst4, *u_chunks, zeros_hbm))


def _tc_prep(deg0, deg1, x):
    def body(d0_ref, d1_ref, x_ref, dinv_ref, u0_ref, u1_ref):
        d = d0_ref[...][:, :1] + d1_ref[...][:, :1]
        dv = jnp.where(d > 0.0, lax.rsqrt(d), 0.0)
        dinv_ref[...] = dv
        xb = x_ref[...]
        u0_ref[...] = xb[:, :128] * dv
        u1_ref[...] = xb[:, 128:] * dv

    return pl.pallas_call(
        body,
        grid=(GRID,),
        in_specs=[
            pl.BlockSpec((R, 128), lambda i: (i, 0)),
            pl.BlockSpec((R, 128), lambda i: (i, 0)),
            pl.BlockSpec((R, 256), lambda i: (i, 0)),
        ],
        out_specs=[
            pl.BlockSpec((R, 1), lambda i: (i, 0)),
            pl.BlockSpec((R, 128), lambda i: (i, 0)),
            pl.BlockSpec((R, 128), lambda i: (i, 0)),
        ],
        out_shape=[
            jax.ShapeDtypeStruct((N, 1), jnp.float32),
            jax.ShapeDtypeStruct((N, 128), jnp.float32),
            jax.ShapeDtypeStruct((N, 128), jnp.float32),
        ],
    )(deg0, deg1, x)


def _tc_layer(s_chunks, dinv, w, b2, scale_out):
    nin = len(s_chunks)
    kdim, hdim = w.shape
    nout = hdim // 128

    def body(*refs):
        s_refs = refs[:nin]
        dinv_ref, w_ref, b_ref = refs[nin], refs[nin + 1], refs[nin + 2]
        o_refs = refs[nin + 3:]
        dv = dinv_ref[...]
        acc = jnp.zeros((R, hdim), jnp.float32)
        for c in range(nin):
            acc = acc + jnp.dot((s_refs[c][...] * dv).astype(jnp.bfloat16),
                                w_ref[pl.ds(c * 128, 128), :].astype(jnp.bfloat16),
                                preferred_element_type=jnp.float32)
        h = jnp.maximum(acc + b_ref[...], 0.0)
        if scale_out:
            h = h * dv
        for c in range(nout):
            o_refs[c][...] = h[:, c * 128:(c + 1) * 128]

    return pl.pallas_call(
        body,
        grid=(GRID,),
        in_specs=[pl.BlockSpec((R, 128), lambda i: (i, 0))] * nin
        + [
            pl.BlockSpec((R, 1), lambda i: (i, 0)),
            pl.BlockSpec((kdim, hdim), lambda i: (0, 0)),
            pl.BlockSpec((1, hdim), lambda i: (0, 0)),
        ],
        out_specs=[pl.BlockSpec((R, 128), lambda i: (i, 0))] * nout,
        out_shape=[jax.ShapeDtypeStruct((N, 128), jnp.float32)] * nout,
    )(*s_chunks, dinv, w, b2)


def _tc_head(h_chunks, lw1, lb1, lw2, lb2):
    def body(h0, h1, h2, h3, w1_ref, b1_ref, w2_ref, b2_ref, o_ref):
        acc = jnp.zeros((R, 64), jnp.float32)
        for c, href in enumerate((h0, h1, h2, h3)):
            acc = acc + jnp.dot(href[...], w1_ref[pl.ds(c * 128, 128), :],
                                preferred_element_type=jnp.float32)
        z = jnp.maximum(acc + b1_ref[...], 0.0)
        logit = jnp.dot(z, w2_ref[...],
                        preferred_element_type=jnp.float32) + b2_ref[...]
        o_ref[...] = 1.0 / (1.0 + jnp.exp(-logit))

    return pl.pallas_call(
        body,
        grid=(GRID,),
        in_specs=[pl.BlockSpec((R, 128), lambda i: (i, 0))] * 4
        + [
            pl.BlockSpec((512, 64), lambda i: (0, 0)),
            pl.BlockSpec((1, 64), lambda i: (0, 0)),
            pl.BlockSpec((64, 1), lambda i: (0, 0)),
            pl.BlockSpec((1, 1), lambda i: (0, 0)),
        ],
        out_specs=pl.BlockSpec((R, 1), lambda i: (i, 0)),
        out_shape=jax.ShapeDtypeStruct((N, 1), jnp.float32),
    )(*h_chunks, lw1, lb1, lw2, lb2)


def kernel(x, edge_index, W1, b1, W2, b2, W3, b3, W4, b4, W5, b5,
           lw1, lb1, lw2, lb2):
    loops = jnp.arange(N, dtype=jnp.int32)
    src = jnp.concatenate([edge_index[0].astype(jnp.int32), loops])
    dst = jnp.concatenate([edge_index[1].astype(jnp.int32), loops])
    src3, dst3, _ = _pad_edges(src, dst)

    zeros128 = jnp.zeros((NP, 128), jnp.float32)
    ones128 = jnp.ones((EW, 128), jnp.float32)

    deg0, deg1 = _sc_degree(dst3, ones128, zeros128)
    dinv, u0a, u0b = _tc_prep(deg0, deg1, x)

    u = [u0a, u0b]
    ws = [W1, W2, W3, W4, W5]
    bs = [b1, b2, b3, b4, b5]
    for l in range(5):
        s = _sc_aggregate(src3, dst3, u, zeros128)
        u = _tc_layer(s, dinv, ws[l], bs[l].reshape(1, -1), scale_out=(l < 4))

    return _tc_head(u, lw1, lb1.reshape(1, -1), lw2, lb2.reshape(1, 1))


# speedup vs baseline: 1.0187x; 1.0187x over previous
import functools

import jax
import jax.numpy as jnp
from jax import lax
from jax.experimental import pallas as pl
from jax.experimental.pallas import tpu as pltpu
from jax.experimental.pallas import tpu_sc as plsc

N = 10000
NP = 10112
NSUB = 16
RPT = NP // NSUB
EW = 128
R = 400
GRID = N // R


def _mesh():
    return plsc.VectorSubcoreMesh(core_axis_name="c", subcore_axis_name="s")


def _pad_edges(src, dst):
    e_tot = src.shape[0]
    nw = -(-e_tot // (NSUB * EW))
    nw += nw % 2
    ep = NSUB * EW * nw
    pad = ep - e_tot
    pad_src = (jnp.arange(pad, dtype=jnp.int32) * 7919) % N
    pad_dst = N + (jnp.arange(pad, dtype=jnp.int32) % 16)
    src3 = jnp.concatenate([src, pad_src]).reshape(NSUB, nw, EW)
    dst3 = jnp.concatenate([dst, pad_dst]).reshape(NSUB, nw, EW)
    return src3, dst3, nw


def _sc_degree(dst3, ones128, zeros128):
    nw = dst3.shape[1]
    nwh = nw // 2
    dst4 = dst3.reshape(NSUB, 2, nwh, EW)

    @functools.partial(
        pl.kernel,
        mesh=_mesh(),
        out_type=[jax.ShapeDtypeStruct((NP, 128), jnp.float32)] * 2,
        scratch_types=[
            pltpu.VMEM((nwh, EW), jnp.int32),
            pltpu.VMEM((EW, 128), jnp.float32),
            pltpu.VMEM_SHARED((NP, 128), jnp.float32),
        ],
    )
    def k(dst_hbm, ones_hbm, zeros_hbm, out0_hbm, out1_hbm, dst_v, ones_v, acc):
        core = lax.axis_index("c")
        sub = lax.axis_index("s")
        row0 = sub * RPT

        def half(half_idx, out_hbm):
            pltpu.sync_copy(dst_hbm.at[sub, half_idx], dst_v)
            pltpu.sync_copy(ones_hbm, ones_v)
            pltpu.sync_copy(zeros_hbm.at[pl.ds(row0, RPT)],
                            acc.at[pl.ds(row0, RPT)])
            plsc.subcore_barrier()

            def body(w, carry):
                pltpu.sync_copy(ones_v, acc.at[dst_v.at[w]], add=True)
                return carry

            lax.fori_loop(0, nwh, body, 0)
            plsc.subcore_barrier()
            pltpu.sync_copy(acc.at[pl.ds(row0, RPT)],
                            out_hbm.at[pl.ds(row0, RPT)])

        @pl.when(core == 0)
        def _():
            half(0, out0_hbm)

        @pl.when(core == 1)
        def _():
            half(1, out1_hbm)

    return k(dst4, ones128, zeros128)


def _sc_aggregate(src3, dst3, u_chunks, zeros_hbm):
    nch = len(u_chunks)
    cpc = nch // 2
    nw = src3.shape[1]
    ngrp = 2
    wpg = nw // ngrp
    src4 = src3.reshape(NSUB, ngrp, wpg, EW)
    dst4 = dst3.reshape(NSUB, ngrp, wpg, EW)

    @functools.partial(
        pl.kernel,
        mesh=_mesh(),
        out_type=[jax.ShapeDtypeStruct((NP, 128), jnp.float32)] * nch,
        scratch_types=[
            pltpu.VMEM((wpg, EW), jnp.int32),
            pltpu.VMEM((wpg, EW), jnp.int32),
            pltpu.VMEM((2, EW, 128), jnp.float32),
            pltpu.VMEM_SHARED((NP, 128), jnp.float32),
            pltpu.SemaphoreType.DMA,
            pltpu.SemaphoreType.DMA,
        ],
    )
    def k(*refs):
        src_hbm, dst_hbm = refs[0], refs[1]
        u_refs = refs[2:2 + nch]
        z_hbm = refs[2 + nch]
        out_refs = refs[3 + nch:3 + 2 * nch]
        src_v, dst_v, rows, acc, sem0, sem1 = refs[3 + 2 * nch:]
        sems = (sem0, sem1)
        core = lax.axis_index("c")
        sub = lax.axis_index("s")
        row0 = sub * RPT

        def one_chunk(u_hbm, out_hbm):
            pltpu.sync_copy(z_hbm.at[pl.ds(row0, RPT)],
                            acc.at[pl.ds(row0, RPT)])
            plsc.subcore_barrier()

            def g_start(w, b):
                pltpu.async_copy(u_hbm.at[src_v.at[w]], rows.at[b], sems[b])

            def g_wait(b):
                pltpu.make_async_copy(u_hbm.at[src_v.at[0]], rows.at[b],
                                      sems[b]).wait()

            for g in range(ngrp):
                pltpu.sync_copy(src_hbm.at[sub, g], src_v)
                pltpu.sync_copy(dst_hbm.at[sub, g], dst_v)
                g_start(0, 0)
                g_start(1, 1)

                def body(i, carry):
                    for b in range(2):
                        w = 2 * i + b
                        g_wait(b)
                        pltpu.sync_copy(rows.at[b], acc.at[dst_v.at[w]],
                                        add=True)

                        @pl.when(w + 2 < wpg)
                        def _():
                            g_start(w + 2, b)
                    return carry

                lax.fori_loop(0, wpg // 2, body, 0)
            plsc.subcore_barrier()
            pltpu.sync_copy(acc.at[pl.ds(row0, RPT)],
                            out_hbm.at[pl.ds(row0, RPT)])

        for p in range(cpc):
            @pl.when(core == 0)
            def _(p=p):
                one_chunk(u_refs[p], out_refs[p])

            @pl.when(core == 1)
            def _(p=p):
                one_chunk(u_refs[cpc + p], out_refs[cpc + p])

    return list(k(src4, dst4, *u_chunks, zeros_hbm))


def _tc_prep(deg0, deg1, x):
    def body(d0_ref, d1_ref, x_ref, dinv_ref, u0_ref, u1_ref):
        d = d0_ref[...][:, :1] + d1_ref[...][:, :1]
        dv = jnp.where(d > 0.0, lax.rsqrt(d), 0.0)
        dinv_ref[...] = dv
        xb = x_ref[...]
        u0_ref[...] = xb[:, :128] * dv
        u1_ref[...] = xb[:, 128:] * dv

    return pl.pallas_call(
        body,
        grid=(GRID,),
        in_specs=[
            pl.BlockSpec((R, 128), lambda i: (i, 0)),
            pl.BlockSpec((R, 128), lambda i: (i, 0)),
            pl.BlockSpec((R, 256), lambda i: (i, 0)),
        ],
        out_specs=[
            pl.BlockSpec((R, 1), lambda i: (i, 0)),
            pl.BlockSpec((R, 128), lambda i: (i, 0)),
            pl.BlockSpec((R, 128), lambda i: (i, 0)),
        ],
        out_shape=[
            jax.ShapeDtypeStruct((N, 1), jnp.float32),
            jax.ShapeDtypeStruct((N, 128), jnp.float32),
            jax.ShapeDtypeStruct((N, 128), jnp.float32),
        ],
    )(deg0, deg1, x)


def _tc_layer(s_chunks, dinv, w, b2, scale_out):
    nin = len(s_chunks)
    kdim, hdim = w.shape
    nout = hdim // 128

    def body(*refs):
        s_refs = refs[:nin]
        dinv_ref, w_ref, b_ref = refs[nin], refs[nin + 1], refs[nin + 2]
        o_refs = refs[nin + 3:]
        dv = dinv_ref[...]
        acc = jnp.zeros((R, hdim), jnp.float32)
        for c in range(nin):
            acc = acc + jnp.dot(s_refs[c][...] * dv,
                                w_ref[pl.ds(c * 128, 128), :],
                                preferred_element_type=jnp.float32)
        h = jnp.maximum(acc + b_ref[...], 0.0)
        if scale_out:
            h = h * dv
        for c in range(nout):
            o_refs[c][...] = h[:, c * 128:(c + 1) * 128]

    return pl.pallas_call(
        body,
        grid=(GRID,),
        in_specs=[pl.BlockSpec((R, 128), lambda i: (i, 0))] * nin
        + [
            pl.BlockSpec((R, 1), lambda i: (i, 0)),
            pl.BlockSpec((kdim, hdim), lambda i: (0, 0)),
            pl.BlockSpec((1, hdim), lambda i: (0, 0)),
        ],
        out_specs=[pl.BlockSpec((R, 128), lambda i: (i, 0))] * nout,
        out_shape=[jax.ShapeDtypeStruct((N, 128), jnp.float32)] * nout,
    )(*s_chunks, dinv, w, b2)


def _tc_last(s_chunks, dinv, w, b2, lw1, lb1, lw2, lb2):
    kdim, hdim = w.shape

    def body(s0, s1, s2, s3, dinv_ref, w_ref, b_ref,
             w1_ref, b1_ref, w2_ref, b2_ref, o_ref):
        dv = dinv_ref[...]
        acc = jnp.zeros((R, hdim), jnp.float32)
        for c, sref in enumerate((s0, s1, s2, s3)):
            acc = acc + jnp.dot(sref[...] * dv,
                                w_ref[pl.ds(c * 128, 128), :],
                                preferred_element_type=jnp.float32)
        h = jnp.maximum(acc + b_ref[...], 0.0)
        z = jnp.maximum(jnp.dot(h, w1_ref[...],
                                preferred_element_type=jnp.float32)
                        + b1_ref[...], 0.0)
        logit = jnp.dot(z, w2_ref[...],
                        preferred_element_type=jnp.float32) + b2_ref[...]
        o_ref[...] = 1.0 / (1.0 + jnp.exp(-logit))

    return pl.pallas_call(
        body,
        grid=(GRID,),
        in_specs=[pl.BlockSpec((R, 128), lambda i: (i, 0))] * 4
        + [
            pl.BlockSpec((R, 1), lambda i: (i, 0)),
            pl.BlockSpec((kdim, hdim), lambda i: (0, 0)),
            pl.BlockSpec((1, hdim), lambda i: (0, 0)),
            pl.BlockSpec((512, 64), lambda i: (0, 0)),
            pl.BlockSpec((1, 64), lambda i: (0, 0)),
            pl.BlockSpec((64, 1), lambda i: (0, 0)),
            pl.BlockSpec((1, 1), lambda i: (0, 0)),
        ],
        out_specs=pl.BlockSpec((R, 1), lambda i: (i, 0)),
        out_shape=jax.ShapeDtypeStruct((N, 1), jnp.float32),
    )(*s_chunks, dinv, w, b2, lw1, lb1, lw2, lb2)


def kernel(x, edge_index, W1, b1, W2, b2, W3, b3, W4, b4, W5, b5,
           lw1, lb1, lw2, lb2):
    loops = jnp.arange(N, dtype=jnp.int32)
    src = jnp.concatenate([edge_index[0].astype(jnp.int32), loops])
    dst = jnp.concatenate([edge_index[1].astype(jnp.int32), loops])
    src3, dst3, _ = _pad_edges(src, dst)

    zeros128 = jnp.zeros((NP, 128), jnp.float32)
    ones128 = jnp.ones((EW, 128), jnp.float32)

    deg0, deg1 = _sc_degree(dst3, ones128, zeros128)
    dinv, u0a, u0b = _tc_prep(deg0, deg1, x)

    u = [u0a, u0b]
    ws = [W1, W2, W3, W4]
    bs = [b1, b2, b3, b4]
    for l in range(4):
        s = _sc_aggregate(src3, dst3, u, zeros128)
        u = _tc_layer(s, dinv, ws[l], bs[l].reshape(1, -1), scale_out=True)

    s = _sc_aggregate(src3, dst3, u, zeros128)
    return _tc_last(s, dinv, W5, b5.reshape(1, -1),
                    lw1, lb1.reshape(1, -1), lw2, lb2.reshape(1, 1))
